# Initial kernel scaffold; baseline (speedup 1.0000x reference)
import jax
import jax.numpy as jnp
from jax.experimental import pallas as pl


def kernel(x, target):
    raise NotImplementedError("write your pallas kernel here")


# speedup vs baseline: 7.5106x; 7.5106x over previous
import functools
import math

import jax
import jax.numpy as jnp
from jax.experimental import pallas as pl
from jax.experimental.pallas import tpu as pltpu

_V = 32000
_PAD = 0
_SMOOTHING = 0.1
_CONF = 1.0 - _SMOOTHING
_EPS = _SMOOTHING / (_V - 1)
_CS = _CONF - _EPS
_C0 = (_V - 1) * _EPS * math.log(_EPS) + _CONF * math.log(_CONF)

_R = 256
_C = 6400


def _body(nc, inv_denom, x_ref, t_ref, out_ref, m_ref, s_ref, sx_ref, xt_ref):
    i = pl.program_id(0)
    j = pl.program_id(1)
    xb = x_ref[...]
    t = t_ref[...]

    bmax = jnp.max(xb, axis=1, keepdims=True)
    bsum = jnp.sum(xb, axis=1, keepdims=True)

    ids = j * _C + jax.lax.broadcasted_iota(jnp.int32, (_R, _C), 1)
    hit = ids == t
    xt_part = jnp.sum(jnp.where(hit, xb, 0.0), axis=1, keepdims=True)

    first = j == 0
    neg_inf = jnp.full((_R, 1), -jnp.inf, dtype=jnp.float32)
    zeros = jnp.zeros((_R, 1), dtype=jnp.float32)
    m_old = jnp.where(first, neg_inf, m_ref[...])
    s_old = jnp.where(first, zeros, s_ref[...])
    sx_old = jnp.where(first, zeros, sx_ref[...])
    xt_old = jnp.where(first, zeros, xt_ref[...])

    m_new = jnp.maximum(m_old, bmax)
    s_new = s_old * jnp.exp(m_old - m_new) + jnp.sum(
        jnp.exp(xb - m_new), axis=1, keepdims=True)
    m_ref[...] = m_new
    s_ref[...] = s_new
    sx_ref[...] = sx_old + bsum
    xt_ref[...] = xt_old + xt_part

    @pl.when(j == nc - 1)
    def _():
        lse = m_new + jnp.log(s_new)
        row_loss = _C0 + lse - _EPS * sx_ref[...] - _CS * xt_ref[...]
        valid = t != _PAD
        contrib = jnp.sum(jnp.where(valid, row_loss, 0.0)) * inv_denom
        prev = jnp.where(i == 0, jnp.zeros((1, 1), jnp.float32), out_ref[...])
        out_ref[...] = prev + contrib


def kernel(x, target):
    batch = x.shape[0]
    n = x.shape[0] * x.shape[1]
    xf = x.reshape(n, _V)
    t = target.reshape(n, 1).astype(jnp.int32)
    nr = n // _R
    nc = _V // _C
    out = pl.pallas_call(
        functools.partial(_body, nc, 1.0 / batch),
        grid=(nr, nc),
        in_specs=[
            pl.BlockSpec((_R, _C), lambda i, j: (i, j)),
            pl.BlockSpec((_R, 1), lambda i, j: (i, 0)),
        ],
        out_specs=pl.BlockSpec((1, 1), lambda i, j: (0, 0)),
        out_shape=jax.ShapeDtypeStruct((1, 1), jnp.float32),
        scratch_shapes=[
            pltpu.VMEM((_R, 1), jnp.float32),
            pltpu.VMEM((_R, 1), jnp.float32),
            pltpu.VMEM((_R, 1), jnp.float32),
            pltpu.VMEM((_R, 1), jnp.float32),
        ],
        compiler_params=pltpu.CompilerParams(
            dimension_semantics=("arbitrary", "arbitrary"),
        ),
    )(xf, t)
    return out[0, 0]


# speedup vs baseline: 8.6073x; 1.1460x over previous
import functools
import math

import jax
import jax.numpy as jnp
from jax.experimental import pallas as pl
from jax.experimental.pallas import tpu as pltpu

_V = 32000
_PAD = 0
_SMOOTHING = 0.1
_CONF = 1.0 - _SMOOTHING
_EPS = _SMOOTHING / (_V - 1)
_CS = _CONF - _EPS
_C0 = (_V - 1) * _EPS * math.log(_EPS) + _CONF * math.log(_CONF)

_R = 256
_C = 16000


def _body(nc, inv_denom, x_ref, t_ref, out_ref, m_ref, s_ref, sx_ref, xt_ref):
    i = pl.program_id(0)
    j = pl.program_id(1)
    xb = x_ref[...]
    t = t_ref[...]

    bmax = jnp.max(xb, axis=1, keepdims=True)
    bsum = jnp.sum(xb, axis=1, keepdims=True)

    ids = j * _C + jax.lax.broadcasted_iota(jnp.int32, (_R, _C), 1)
    hit = ids == t
    xt_part = jnp.sum(jnp.where(hit, xb, 0.0), axis=1, keepdims=True)

    first = j == 0
    neg_inf = jnp.full((_R, 1), -jnp.inf, dtype=jnp.float32)
    zeros = jnp.zeros((_R, 1), dtype=jnp.float32)
    m_old = jnp.where(first, neg_inf, m_ref[...])
    s_old = jnp.where(first, zeros, s_ref[...])
    sx_old = jnp.where(first, zeros, sx_ref[...])
    xt_old = jnp.where(first, zeros, xt_ref[...])

    m_new = jnp.maximum(m_old, bmax)
    s_new = s_old * jnp.exp(m_old - m_new) + jnp.sum(
        jnp.exp(xb - m_new), axis=1, keepdims=True)
    m_ref[...] = m_new
    s_ref[...] = s_new
    sx_ref[...] = sx_old + bsum
    xt_ref[...] = xt_old + xt_part

    @pl.when(j == nc - 1)
    def _():
        lse = m_new + jnp.log(s_new)
        row_loss = _C0 + lse - _EPS * sx_ref[...] - _CS * xt_ref[...]
        valid = t != _PAD
        contrib = jnp.sum(jnp.where(valid, row_loss, 0.0)) * inv_denom
        prev = jnp.where(i == 0, jnp.zeros((1, 1), jnp.float32), out_ref[...])
        out_ref[...] = prev + contrib


def kernel(x, target):
    batch = x.shape[0]
    n = x.shape[0] * x.shape[1]
    xf = x.reshape(n, _V)
    t = target.reshape(n, 1).astype(jnp.int32)
    nr = n // _R
    nc = _V // _C
    out = pl.pallas_call(
        functools.partial(_body, nc, 1.0 / batch),
        grid=(nr, nc),
        in_specs=[
            pl.BlockSpec((_R, _C), lambda i, j: (i, j)),
            pl.BlockSpec((_R, 1), lambda i, j: (i, 0)),
        ],
        out_specs=pl.BlockSpec((1, 1), lambda i, j: (0, 0)),
        out_shape=jax.ShapeDtypeStruct((1, 1), jnp.float32),
        scratch_shapes=[
            pltpu.VMEM((_R, 1), jnp.float32),
            pltpu.VMEM((_R, 1), jnp.float32),
            pltpu.VMEM((_R, 1), jnp.float32),
            pltpu.VMEM((_R, 1), jnp.float32),
        ],
        compiler_params=pltpu.CompilerParams(
            dimension_semantics=("arbitrary", "arbitrary"),
        ),
    )(xf, t)
    return out[0, 0]
